# baseline (speedup 1.0000x reference)
import functools

import jax
import jax.numpy as jnp
from jax import lax
from jax.experimental import pallas as pl
from jax.experimental.pallas import tpu as pltpu
from jax.experimental.pallas import tpu_sc as plsc

_NC = 2
_NS = 16
_NW = _NC * _NS


def _topk_body(x_ref, nk_ref, idx_ref):
    x = x_ref[...]
    q = x / jnp.maximum(jnp.sqrt(jnp.sum(x * x, axis=1, keepdims=True)),
                        1e-12)
    scores = lax.dot_general(q, nk_ref[...], (((1,), (1,)), ((), ())),
                             preferred_element_type=jnp.float32)
    m = jnp.max(scores, axis=1, keepdims=True)
    col = lax.broadcasted_iota(jnp.int32, scores.shape, 1)
    idx_ref[...] = jnp.min(jnp.where(scores == m, col, jnp.int32(2**30)),
                           axis=1)


def _topk_indices(x, nk):
    b = x.shape[0]
    return pl.pallas_call(
        _topk_body,
        out_shape=jax.ShapeDtypeStruct((b,), jnp.int32),
    )(x, nk)


def _make_gather_spmem(b, half, emb, pool, plen):
    bpw = b // _NW
    mesh = plsc.VectorSubcoreMesh(core_axis_name="c", subcore_axis_name="s")

    @functools.partial(
        pl.kernel,
        mesh=mesh,
        out_type=jax.ShapeDtypeStruct((b, half, emb), jnp.float32),
        scratch_types=[
            pltpu.VMEM((bpw,), jnp.int32),
            pltpu.VMEM_SHARED((pool, plen, emb), jnp.float32),
            pltpu.SemaphoreType.DMA,
        ],
    )
    def gather_kernel(ep_hbm, idx_hbm, ek_out, idx_v, shared, sem):
        c = lax.axis_index("c")
        s = lax.axis_index("s")
        wid = s * _NC + c
        base = wid * bpw

        @pl.when(s == 0)
        def _():
            pltpu.sync_copy(ep_hbm, shared)

        pltpu.sync_copy(idx_hbm.at[pl.ds(base, bpw)], idx_v)
        plsc.subcore_barrier()

        def body(g, carry):
            off = pl.multiple_of(g * 16, 16)
            vec = idx_v[pl.ds(off, 16)]
            for j in range(16):
                pltpu.async_copy(
                    shared.at[pl.ds(vec[j], 1), pl.ds(0, half)],
                    ek_out.at[pl.ds(base + off + j, 1)], sem)
            return carry

        lax.fori_loop(0, bpw // 16, body, 0)
        pltpu.make_async_copy(ek_out.at[pl.ds(base, bpw)],
                              ek_out.at[pl.ds(base, bpw)], sem).wait()

    return gather_kernel


def _make_gather_tc(b, half, emb):
    blk = 512

    def body(idx_ref, ep_ref, out_ref):
        amax = idx_ref[...]
        pool = ep_ref.shape[0]
        col = lax.broadcasted_iota(jnp.int32, (blk, pool), 1)
        oh = (col == amax[:, None]).astype(jnp.float32)
        for h in range(half):
            out_ref[:, h, :] = lax.dot_general(
                oh, ep_ref[:, half + h, :], (((1,), (0,)), ((), ())),
                preferred_element_type=jnp.float32,
                precision=lax.Precision.HIGHEST)

    def call(amax, e_p):
        pool, plen, _ = e_p.shape
        return pl.pallas_call(
            body,
            grid=(b // blk,),
            in_specs=[pl.BlockSpec((blk,), lambda g: (g,)),
                      pl.BlockSpec((pool, plen, emb), lambda g: (0, 0, 0))],
            out_specs=pl.BlockSpec((blk, half, emb), lambda g: (g, 0, 0)),
            out_shape=jax.ShapeDtypeStruct((b, half, emb), jnp.float32),
        )(amax, e_p)

    return call


def kernel(x_querry, l, x_block, e_k, e_p):
    b = x_querry.shape[0]
    pool, plen, emb = e_p.shape
    half = plen // 2
    d = half * emb

    nk = e_k / jnp.maximum(jnp.linalg.norm(e_k, axis=1, keepdims=True), 1e-12)
    amax = _topk_indices(x_querry, nk)
    ek_o = _make_gather_spmem(b, half, emb, pool, plen)(e_p, amax)
    ev_o = _make_gather_tc(b, half, emb)(amax, e_p)
    return (ek_o, ev_o, x_block)


# speedup vs baseline: 1.2114x; 1.2114x over previous
import functools

import jax
import jax.numpy as jnp
from jax import lax
from jax.experimental import pallas as pl
from jax.experimental.pallas import tpu as pltpu
from jax.experimental.pallas import tpu_sc as plsc

_NC = 2
_NS = 16
_NW = _NC * _NS


def _topk_body(x_ref, nk_ref, idx_ref):
    x = x_ref[...]
    q = x / jnp.maximum(jnp.sqrt(jnp.sum(x * x, axis=1, keepdims=True)),
                        1e-12)
    scores = lax.dot_general(q, nk_ref[...], (((1,), (1,)), ((), ())),
                             preferred_element_type=jnp.float32)
    m = jnp.max(scores, axis=1, keepdims=True)
    col = lax.broadcasted_iota(jnp.int32, scores.shape, 1)
    idx_ref[...] = jnp.min(jnp.where(scores == m, col, jnp.int32(2**30)),
                           axis=1)


def _topk_indices(x, nk):
    b = x.shape[0]
    return pl.pallas_call(
        _topk_body,
        out_shape=jax.ShapeDtypeStruct((b,), jnp.int32),
    )(x, nk)


def _make_gather_spmem(b, half, emb, pool, plen):
    bpw = b // _NW
    mesh = plsc.VectorSubcoreMesh(core_axis_name="c", subcore_axis_name="s")

    @functools.partial(
        pl.kernel,
        mesh=mesh,
        out_type=jax.ShapeDtypeStruct((b, half, emb), jnp.float32),
        scratch_types=[
            pltpu.VMEM((bpw,), jnp.int32),
            pltpu.VMEM_SHARED((pool, plen, emb), jnp.float32),
            pltpu.SemaphoreType.DMA,
        ],
    )
    def gather_kernel(ep_hbm, idx_hbm, ek_out, idx_v, shared, sem):
        c = lax.axis_index("c")
        s = lax.axis_index("s")
        wid = s * _NC + c
        base = wid * bpw

        @pl.when(s == 0)
        def _():
            pltpu.sync_copy(ep_hbm, shared)

        pltpu.sync_copy(idx_hbm.at[pl.ds(base, bpw)], idx_v)
        plsc.subcore_barrier()

        def body(g, carry):
            off = pl.multiple_of(g * 16, 16)
            vec = idx_v[pl.ds(off, 16)]
            for j in range(16):
                pltpu.async_copy(
                    shared.at[pl.ds(vec[j], 1), pl.ds(0, half)],
                    ek_out.at[pl.ds(base + off + j, 1)], sem)
            return carry

        lax.fori_loop(0, bpw // 16, body, 0)
        pltpu.make_async_copy(ek_out.at[pl.ds(base, bpw)],
                              ek_out.at[pl.ds(base, bpw)], sem).wait()

    return gather_kernel


def _make_gather_tc(b, half, emb):
    blk = 512

    def body(idx_ref, ep_ref, out_ref):
        amax = idx_ref[...]
        pool = ep_ref.shape[0]
        col = lax.broadcasted_iota(jnp.int32, (blk, pool), 1)
        oh = (col == amax[:, None]).astype(jnp.float32)
        for h in range(half):
            out_ref[:, h, :] = lax.dot_general(
                oh, ep_ref[:, half + h, :], (((1,), (0,)), ((), ())),
                preferred_element_type=jnp.float32)

    def call(amax, e_p):
        pool, plen, _ = e_p.shape
        return pl.pallas_call(
            body,
            grid=(b // blk,),
            in_specs=[pl.BlockSpec((blk,), lambda g: (g,)),
                      pl.BlockSpec((pool, plen, emb), lambda g: (0, 0, 0))],
            out_specs=pl.BlockSpec((blk, half, emb), lambda g: (g, 0, 0)),
            out_shape=jax.ShapeDtypeStruct((b, half, emb), jnp.float32),
        )(amax, e_p)

    return call


def kernel(x_querry, l, x_block, e_k, e_p):
    b = x_querry.shape[0]
    pool, plen, emb = e_p.shape
    half = plen // 2
    d = half * emb

    nk = e_k / jnp.maximum(jnp.linalg.norm(e_k, axis=1, keepdims=True), 1e-12)
    amax = _topk_indices(x_querry, nk)
    ek_o = _make_gather_spmem(b, half, emb, pool, plen)(e_p, amax)
    ev_o = _make_gather_tc(b, half, emb)(amax, e_p)
    return (ek_o, ev_o, x_block)
